# baseline (speedup 1.0000x reference)
import functools

import jax
import jax.numpy as jnp
from jax import lax
from jax.experimental import pallas as pl
from jax.experimental.pallas import tpu as pltpu
from jax.experimental.pallas import tpu_sc as plsc

N = 10000
E = 320000
D = 128

NC = 2
NS = 16
NW = NC * NS

C = 64
NBUF = 4
BLK = 4
NBLK = 40
CHUNKS = BLK * NBLK
EP_TILE = CHUNKS * C
E_PAD = NW * EP_TILE

N_PAD = 10112
ROWS_TILE = N_PAD // NS

NCOMB = 18
H_USED = N_PAD * NCOMB
H_SIZE = 182272
H_TILE = H_SIZE // NS


def _sc_body(node_hbm, src_hbm, dst_hbm, eidx_hbm, z2_hbm, z1_hbm,
             agg_out, hist_out,
             src_b0, src_b1, dst_b0, dst_b1, eidx_b0, eidx_b1,
             rows0, rows1, rows2, rows3, ones_v, agg_s, hist_s,
             sem_i0, sem_i1, sem_g0, sem_g1, sem_g2, sem_g3,
             sem_s0, sem_s1, sem_s2, sem_s3, sem_h):
    cid = lax.axis_index("c")
    sid = lax.axis_index("s")
    wid = sid * NC + cid
    srcb = (src_b0, src_b1)
    dstb = (dst_b0, dst_b1)
    eidxb = (eidx_b0, eidx_b1)
    semi = (sem_i0, sem_i1)
    semg = (sem_g0, sem_g1, sem_g2, sem_g3)
    sems = (sem_s0, sem_s1, sem_s2, sem_s3)
    rows = (rows0, rows1, rows2, rows3)

    pltpu.sync_copy(z2_hbm, agg_s.at[pl.ds(sid * ROWS_TILE, ROWS_TILE)])
    pltpu.sync_copy(z1_hbm, hist_s.at[pl.ds(sid * H_TILE, H_TILE)])

    for c in range(C // 16):
        ones_v[pl.ds(c * 16, 16)] = jnp.ones((16,), jnp.float32)

    plsc.subcore_barrier()

    def issue_idx(b, s):
        sl = pl.ds(b * BLK, BLK)
        pltpu.async_copy(src_hbm.at[wid, sl], srcb[s], semi[s])
        pltpu.async_copy(dst_hbm.at[wid, sl], dstb[s], semi[s])
        pltpu.async_copy(eidx_hbm.at[wid, sl], eidxb[s], semi[s])

    def wait_idx(s):
        sl = pl.ds(0, BLK)
        pltpu.make_async_copy(src_hbm.at[0, sl], srcb[s], semi[s]).wait()
        pltpu.make_async_copy(dst_hbm.at[0, sl], dstb[s], semi[s]).wait()
        pltpu.make_async_copy(eidx_hbm.at[0, sl], eidxb[s], semi[s]).wait()

    issue_idx(0, 0)

    def pair_body(base, carry):
        for s in (0, 1):
            b = 2 * base + s
            wait_idx(s)

            @pl.when(b + 1 < NBLK)
            def _():
                issue_idx(b + 1, 1 - s)

            def gather(k, buf):
                return pltpu.async_copy(
                    node_hbm.at[srcb[s].at[k]], rows[buf], semg[buf])

            def scat(k, buf):
                a = pltpu.async_copy(
                    rows[buf], agg_s.at[dstb[s].at[k]], sems[buf], add=True)
                pltpu.async_copy(
                    ones_v, hist_s.at[eidxb[s].at[k]], sem_h, add=True)
                return a

            g = [gather(k, k) for k in range(NBUF)]
            sc = [None] * NBUF
            for k in range(NBUF):
                g[k].wait()
                sc[k] = scat(k, k)
            for k in range(NBUF, BLK):
                buf = k % NBUF
                sc[buf].wait()
                g[buf] = gather(k, buf)
            for k in range(NBUF, BLK):
                buf = k % NBUF
                g[buf].wait()
                sc[buf] = scat(k, buf)
            for buf in range(NBUF):
                sc[buf].wait()
            for _ in range(BLK):
                pltpu.make_async_copy(
                    ones_v, hist_s.at[eidxb[s].at[0]], sem_h).wait()
        return carry

    lax.fori_loop(0, NBLK // 2, pair_body, 0)

    plsc.subcore_barrier()

    pltpu.sync_copy(agg_s.at[pl.ds(sid * ROWS_TILE, ROWS_TILE)],
                    agg_out.at[cid, pl.ds(sid * ROWS_TILE, ROWS_TILE)])
    pltpu.sync_copy(hist_s.at[pl.ds(sid * H_TILE, H_TILE)],
                    hist_out.at[cid, pl.ds(sid * H_TILE, H_TILE)])


_sc_edge_agg = functools.partial(
    pl.kernel,
    out_type=(
        jax.ShapeDtypeStruct((NC, N_PAD, D), jnp.float32),
        jax.ShapeDtypeStruct((NC, H_SIZE), jnp.float32),
    ),
    mesh=plsc.VectorSubcoreMesh(core_axis_name="c", subcore_axis_name="s"),
    scratch_types=[
        pltpu.VMEM((BLK, C), jnp.int32),
        pltpu.VMEM((BLK, C), jnp.int32),
        pltpu.VMEM((BLK, C), jnp.int32),
        pltpu.VMEM((BLK, C), jnp.int32),
        pltpu.VMEM((BLK, C), jnp.int32),
        pltpu.VMEM((BLK, C), jnp.int32),
        pltpu.VMEM((C, D), jnp.float32),
        pltpu.VMEM((C, D), jnp.float32),
        pltpu.VMEM((C, D), jnp.float32),
        pltpu.VMEM((C, D), jnp.float32),
        pltpu.VMEM((C,), jnp.float32),
        pltpu.VMEM_SHARED((N_PAD, D), jnp.float32),
        pltpu.VMEM_SHARED((H_SIZE,), jnp.float32),
        pltpu.SemaphoreType.DMA,
        pltpu.SemaphoreType.DMA,
        pltpu.SemaphoreType.DMA,
        pltpu.SemaphoreType.DMA,
        pltpu.SemaphoreType.DMA,
        pltpu.SemaphoreType.DMA,
        pltpu.SemaphoreType.DMA,
        pltpu.SemaphoreType.DMA,
        pltpu.SemaphoreType.DMA,
        pltpu.SemaphoreType.DMA,
        pltpu.SemaphoreType.DMA,
    ],
)(_sc_body)


def _tc_body(agg_ref, hist_ref, ecomb_ref, w1_ref, b1_ref, w2_ref, b2_ref,
             gamma_ref, beta_ref, out_ref):
    hist = hist_ref[0, :N, :] + hist_ref[1, :N, :]
    agg = agg_ref[0, :N, :] + agg_ref[1, :N, :]
    agg = agg + jnp.dot(hist, ecomb_ref[...], preferred_element_type=jnp.float32)
    h = jnp.dot(agg, w1_ref[...], preferred_element_type=jnp.float32) + b1_ref[...]
    h = jnp.maximum(h, 0.0)
    h = jnp.dot(h, w2_ref[...], preferred_element_type=jnp.float32) + b2_ref[...]
    mean = jnp.mean(h, axis=0, keepdims=True)
    var = jnp.mean((h - mean) ** 2, axis=0, keepdims=True)
    out_ref[...] = (h - mean) * lax.rsqrt(var + 1e-5) * gamma_ref[...] + beta_ref[...]


_tc_mlp = pl.pallas_call(
    _tc_body,
    out_shape=jax.ShapeDtypeStruct((N, D), jnp.float32),
)


def kernel(node_feats, edge_index, edge_feat0, edge_feat1,
           emb0, emb1, W1, b1, W2, b2, gamma, beta):
    node_feats = node_feats.astype(jnp.float32)
    src = edge_index[0].astype(jnp.int32)
    dst = edge_index[1].astype(jnp.int32)
    f0 = edge_feat0.astype(jnp.int32)
    f1 = edge_feat1.astype(jnp.int32)

    pad = E_PAD - E
    dummy_dst = N + (jnp.arange(pad, dtype=jnp.int32) % (N_PAD - N))
    dummy_src = jnp.arange(pad, dtype=jnp.int32) % N
    src_p = jnp.concatenate([src, dummy_src]).reshape(NW, CHUNKS, C)
    dst_full = jnp.concatenate([dst, dummy_dst])
    f0_full = jnp.pad(f0, (0, pad))
    f1_full = jnp.pad(f1, (0, pad))
    eidx_p = (dst_full * NCOMB + f0_full * 3 + f1_full).reshape(NW, CHUNKS, C)
    dst_p = dst_full.reshape(NW, CHUNKS, C)

    z2 = jnp.zeros((ROWS_TILE, D), jnp.float32)
    z1 = jnp.zeros((H_TILE,), jnp.float32)

    agg2, hist2 = _sc_edge_agg(node_feats, src_p, dst_p, eidx_p, z2, z1)
    hist3 = hist2[:, :H_USED].reshape(NC, N_PAD, NCOMB)

    ecomb = (emb0[:, None, :] + emb1[None, :, :]).reshape(NCOMB, D)

    return _tc_mlp(agg2, hist3, ecomb, W1, b1.reshape(1, 2 * D), W2,
                   b2.reshape(1, D), gamma.reshape(1, D), beta.reshape(1, D))


# speedup vs baseline: 1.0858x; 1.0858x over previous
import functools

import jax
import jax.numpy as jnp
from jax import lax
from jax.experimental import pallas as pl
from jax.experimental.pallas import tpu as pltpu
from jax.experimental.pallas import tpu_sc as plsc

N = 10000
E = 320000
D = 128

NC = 2
NS = 16
NW = NC * NS

C = 128
BLK = 4
NBLK = 20
CHUNKS = BLK * NBLK
EP_TILE = CHUNKS * C
E_PAD = NW * EP_TILE

N_PAD = 10112
ROWS_TILE = N_PAD // NS

NCOMB = 18
H_USED = N_PAD * NCOMB
H_SIZE = 182272
H_TILE = H_SIZE // NS


def _sc_body(node_hbm, src_hbm, dst_hbm, eidx_hbm, z2_hbm, z1_hbm,
             agg_out, hist_out,
             src_b0, src_b1, dst_b0, dst_b1, eidx_b0, eidx_b1,
             rows0, rows1, ones_v, agg_s, hist_s,
             sem_i0, sem_i1, sem_g0, sem_g1, sem_s0, sem_s1, sem_h):
    cid = lax.axis_index("c")
    sid = lax.axis_index("s")
    wid = sid * NC + cid
    srcb = (src_b0, src_b1)
    dstb = (dst_b0, dst_b1)
    eidxb = (eidx_b0, eidx_b1)
    semi = (sem_i0, sem_i1)
    semg = (sem_g0, sem_g1)
    sems = (sem_s0, sem_s1)
    rows = (rows0, rows1)

    za = pltpu.async_copy(
        z2_hbm, agg_s.at[pl.ds(sid * ROWS_TILE, ROWS_TILE)], sem_h)
    zh = pltpu.async_copy(
        z1_hbm, hist_s.at[pl.ds(sid * H_TILE, H_TILE)], sem_h)
    za.wait()
    zh.wait()

    for c in range(C // 16):
        ones_v[pl.ds(c * 16, 16)] = jnp.ones((16,), jnp.float32)

    plsc.subcore_barrier()

    def issue_idx(b, s):
        sl = pl.ds(b * BLK, BLK)
        pltpu.async_copy(src_hbm.at[wid, sl], srcb[s], semi[s])
        pltpu.async_copy(dst_hbm.at[wid, sl], dstb[s], semi[s])
        pltpu.async_copy(eidx_hbm.at[wid, sl], eidxb[s], semi[s])

    def wait_idx(s):
        sl = pl.ds(0, BLK)
        pltpu.make_async_copy(src_hbm.at[0, sl], srcb[s], semi[s]).wait()
        pltpu.make_async_copy(dst_hbm.at[0, sl], dstb[s], semi[s]).wait()
        pltpu.make_async_copy(eidx_hbm.at[0, sl], eidxb[s], semi[s]).wait()

    issue_idx(0, 0)

    def pair_body(base, carry):
        for s in (0, 1):
            b = 2 * base + s
            wait_idx(s)

            @pl.when(b + 1 < NBLK)
            def _():
                issue_idx(b + 1, 1 - s)

            def gather(k, buf):
                return pltpu.async_copy(
                    node_hbm.at[srcb[s].at[k]], rows[buf], semg[buf])

            def scat(k, buf):
                a = pltpu.async_copy(
                    rows[buf], agg_s.at[dstb[s].at[k]], sems[buf], add=True)
                pltpu.async_copy(
                    ones_v, hist_s.at[eidxb[s].at[k]], sem_h, add=True)
                return a

            g0 = gather(0, 0)
            g1 = gather(1, 1)
            g0.wait()
            s0 = scat(0, 0)
            g1.wait()
            s1 = scat(1, 1)
            s0.wait()
            g2 = gather(2, 0)
            s1.wait()
            g3 = gather(3, 1)
            g2.wait()
            s2 = scat(2, 0)
            g3.wait()
            s3 = scat(3, 1)
            s2.wait()
            s3.wait()
            for _ in range(BLK):
                pltpu.make_async_copy(
                    ones_v, hist_s.at[eidxb[s].at[0]], sem_h).wait()
        return carry

    lax.fori_loop(0, NBLK // 2, pair_body, 0)

    plsc.subcore_barrier()

    wa = pltpu.async_copy(
        agg_s.at[pl.ds(sid * ROWS_TILE, ROWS_TILE)],
        agg_out.at[cid, pl.ds(sid * ROWS_TILE, ROWS_TILE)], sem_h)
    wh = pltpu.async_copy(
        hist_s.at[pl.ds(sid * H_TILE, H_TILE)],
        hist_out.at[cid, pl.ds(sid * H_TILE, H_TILE)], sem_h)
    wa.wait()
    wh.wait()


_sc_edge_agg = functools.partial(
    pl.kernel,
    out_type=(
        jax.ShapeDtypeStruct((NC, N_PAD, D), jnp.float32),
        jax.ShapeDtypeStruct((NC, H_SIZE), jnp.float32),
    ),
    mesh=plsc.VectorSubcoreMesh(core_axis_name="c", subcore_axis_name="s"),
    scratch_types=[
        pltpu.VMEM((BLK, C), jnp.int32),
        pltpu.VMEM((BLK, C), jnp.int32),
        pltpu.VMEM((BLK, C), jnp.int32),
        pltpu.VMEM((BLK, C), jnp.int32),
        pltpu.VMEM((BLK, C), jnp.int32),
        pltpu.VMEM((BLK, C), jnp.int32),
        pltpu.VMEM((C, D), jnp.float32),
        pltpu.VMEM((C, D), jnp.float32),
        pltpu.VMEM((C,), jnp.float32),
        pltpu.VMEM_SHARED((N_PAD, D), jnp.float32),
        pltpu.VMEM_SHARED((H_SIZE,), jnp.float32),
        pltpu.SemaphoreType.DMA,
        pltpu.SemaphoreType.DMA,
        pltpu.SemaphoreType.DMA,
        pltpu.SemaphoreType.DMA,
        pltpu.SemaphoreType.DMA,
        pltpu.SemaphoreType.DMA,
        pltpu.SemaphoreType.DMA,
    ],
)(_sc_body)


def _tc_body(agg_ref, hist_ref, ecomb_ref, w1_ref, b1_ref, w2_ref, b2_ref,
             gamma_ref, beta_ref, out_ref):
    hist = hist_ref[0, :N, :] + hist_ref[1, :N, :]
    agg = agg_ref[0, :N, :] + agg_ref[1, :N, :]
    agg = agg + jnp.dot(hist, ecomb_ref[...], preferred_element_type=jnp.float32)
    h = jnp.dot(agg, w1_ref[...], preferred_element_type=jnp.float32) + b1_ref[...]
    h = jnp.maximum(h, 0.0)
    h = jnp.dot(h, w2_ref[...], preferred_element_type=jnp.float32) + b2_ref[...]
    mean = jnp.mean(h, axis=0, keepdims=True)
    var = jnp.mean((h - mean) ** 2, axis=0, keepdims=True)
    out_ref[...] = (h - mean) * lax.rsqrt(var + 1e-5) * gamma_ref[...] + beta_ref[...]


_tc_mlp = pl.pallas_call(
    _tc_body,
    out_shape=jax.ShapeDtypeStruct((N, D), jnp.float32),
)


def kernel(node_feats, edge_index, edge_feat0, edge_feat1,
           emb0, emb1, W1, b1, W2, b2, gamma, beta):
    node_feats = node_feats.astype(jnp.float32)
    src = edge_index[0].astype(jnp.int32)
    dst = edge_index[1].astype(jnp.int32)
    f0 = edge_feat0.astype(jnp.int32)
    f1 = edge_feat1.astype(jnp.int32)

    pad = E_PAD - E
    dummy_dst = N + (jnp.arange(pad, dtype=jnp.int32) % (N_PAD - N))
    dummy_src = jnp.arange(pad, dtype=jnp.int32) % N
    src_p = jnp.concatenate([src, dummy_src]).reshape(NW, CHUNKS, C)
    dst_full = jnp.concatenate([dst, dummy_dst])
    f0_full = jnp.pad(f0, (0, pad))
    f1_full = jnp.pad(f1, (0, pad))
    eidx_p = (dst_full * NCOMB + f0_full * 3 + f1_full).reshape(NW, CHUNKS, C)
    dst_p = dst_full.reshape(NW, CHUNKS, C)

    z2 = jnp.zeros((ROWS_TILE, D), jnp.float32)
    z1 = jnp.zeros((H_TILE,), jnp.float32)

    agg2, hist2 = _sc_edge_agg(node_feats, src_p, dst_p, eidx_p, z2, z1)
    hist3 = hist2[:, :H_USED].reshape(NC, N_PAD, NCOMB)

    ecomb = (emb0[:, None, :] + emb1[None, :, :]).reshape(NCOMB, D)

    return _tc_mlp(agg2, hist3, ecomb, W1, b1.reshape(1, 2 * D), W2,
                   b2.reshape(1, D), gamma.reshape(1, D), beta.reshape(1, D))
